# baseline (speedup 1.0000x reference)
import functools

import jax
import jax.numpy as jnp
from jax import lax
from jax.experimental import pallas as pl
from jax.experimental.pallas import tpu as pltpu
from jax.experimental.pallas import tpu_sc as plsc

N = 10000
E = 160000
D = 128
K = 5000


_WSC = 1_280_000
_GWIN = _WSC + 128
_NPASS = N * 10240 // (2 * _WSC)
_ECH = E // 16
_WCH = 20_000
_NCH = _WSC // _WCH


def _adj_body(src_hbm, dst_hbm, g_hbm, gwin, keyb, dstc, stage2d,
              ones_v, zeros_v, wbuf):
    c = lax.axis_index("c")
    s = lax.axis_index("s")
    lanes = lax.iota(jnp.int32, 16)

    def _zb(i, carry):
        wbuf[pl.ds(i * 16, 16)] = jnp.zeros((16,), jnp.float32)
        return carry
    lax.fori_loop(0, _WCH // 16, _zb, 0)
    for j in range(8):
        ones_v[pl.ds(j * 16, 16)] = jnp.ones((16,), jnp.float32)
        zeros_v[pl.ds(j * 16, 16)] = jnp.zeros((16,), jnp.float32)

    pltpu.sync_copy(src_hbm.at[pl.ds(s * _ECH, _ECH)], keyb)
    for j in range(5):
        pltpu.sync_copy(dst_hbm.at[pl.ds(s * _ECH + j * 2000, 2000)], dstc)

        def _key(i, carry):
            keyb[pl.ds(j * 2000 + i * 16, 16)] = (
                keyb[pl.ds(j * 2000 + i * 16, 16)] * _NP + dstc[pl.ds(i * 16, 16)])
            return carry
        lax.fori_loop(0, 2000 // 16, _key, 0)

    for j in range(4):
        @pl.when(s + 16 * j < _NCH)
        def _z():
            pltpu.sync_copy(wbuf, gwin.at[pl.ds((s + 16 * j) * _WCH, _WCH)])
    @pl.when(s == 0)
    def _zd():
        pltpu.sync_copy(wbuf.at[pl.ds(0, 128)], gwin.at[pl.ds(_WSC, 128)])
    plsc.subcore_barrier()

    def _pass(p, carry):
        base = p * (2 * _WSC) + c * _WSC

        def _scan(i, cnt):
            k = keyb[pl.ds(i * 16, 16)]
            t = k - base
            m = (t >= 0) & (t < _WSC)
            n_vec = plsc.all_reduce_population_count(m)
            _, tc = plsc.sort_key_val(jnp.where(m, 0, 1), t)
            pos = cnt + lanes
            plsc.store_scatter(stage2d, [pos >> 7, pos & 127], tc)
            return cnt + n_vec[0]
        cnt = lax.fori_loop(0, _ECH // 16, _scan, 0)

        for j in range(8):
            pos = cnt + j * 16 + lanes
            plsc.store_scatter(stage2d, [pos >> 7, pos & 127],
                               _WSC + j * 16 + lanes)
        nb = (cnt + 127) // 128

        def _sc(b, carry):
            pltpu.sync_copy(ones_v, gwin.at[stage2d.at[b]], add=True)
            return carry
        lax.fori_loop(0, nb, _sc, 0)
        plsc.subcore_barrier()

        for j in range(4):
            @pl.when(s + 16 * j < _NCH)
            def _wb():
                off = (s + 16 * j) * _WCH
                pltpu.sync_copy(gwin.at[pl.ds(off, _WCH)], wbuf)
                pltpu.sync_copy(wbuf, g_hbm.at[pl.ds(base + off, _WCH)])
        plsc.subcore_barrier()

        def _rz(b, carry):
            pltpu.sync_copy(zeros_v, gwin.at[stage2d.at[b]])
            return carry
        lax.fori_loop(0, nb, _rz, 0)
        plsc.subcore_barrier()
        return carry

    lax.fori_loop(0, _NPASS, _pass, 0)


@jax.jit
def _adj_counts(src, dst):
    mesh = plsc.VectorSubcoreMesh(core_axis_name="c", subcore_axis_name="s")
    f = pl.kernel(
        _adj_body,
        out_type=jax.ShapeDtypeStruct((N * 10240,), jnp.float32),
        mesh=mesh,
        scratch_types=[
            pltpu.VMEM_SHARED((_GWIN,), jnp.float32),
            pltpu.VMEM((_ECH,), jnp.int32),
            pltpu.VMEM((2000,), jnp.int32),
            pltpu.VMEM((80, 128), jnp.int32),
            pltpu.VMEM((128,), jnp.float32),
            pltpu.VMEM((128,), jnp.float32),
            pltpu.VMEM((_WCH,), jnp.float32),
        ],
        compiler_params=pltpu.CompilerParams(needs_layout_passes=False),
    )
    return f(src, dst)


_BM = 200


def _score_body(adj_ref, h_ref, wd_ref, bd_ref, wp_ref, bp_ref,
                g_ref, h1_ref, sc_ref, inv_ref):
    adjp = adj_ref[...]
    adj = adjp[:, :N]
    deg = jnp.sum(adjp, axis=1, keepdims=True)
    g = adj / (deg + 1e-8)
    g_ref[...] = g
    inv_ref[...] = 1.0 / (deg + 1e-8)
    t = jnp.dot(g, h_ref[...])
    h1 = jnp.maximum(jnp.dot(t, wd_ref[...]) + bd_ref[...], 0.0)
    h1_ref[...] = h1
    sgt = jnp.dot(h1, wp_ref[...]) + bp_ref[...]
    sc_ref[...] = 1.0 / (1.0 + jnp.exp(-sgt))


def _dense_scores(adj, h, W_down, b_down, W_pool, b_pool):
    grid = N // _BM
    return pl.pallas_call(
        _score_body,
        grid=(grid,),
        in_specs=[
            pl.BlockSpec((_BM, 10240), lambda i: (i, 0)),
            pl.BlockSpec((N, D), lambda i: (0, 0)),
            pl.BlockSpec((D, D), lambda i: (0, 0)),
            pl.BlockSpec((1, D), lambda i: (0, 0)),
            pl.BlockSpec((D, 1), lambda i: (0, 0)),
            pl.BlockSpec((1, 1), lambda i: (0, 0)),
        ],
        out_specs=[
            pl.BlockSpec((_BM, N), lambda i: (i, 0)),
            pl.BlockSpec((_BM, D), lambda i: (i, 0)),
            pl.BlockSpec((_BM, 1), lambda i: (i, 0)),
            pl.BlockSpec((_BM, 1), lambda i: (i, 0)),
        ],
        out_shape=[
            jax.ShapeDtypeStruct((N, N), jnp.float32),
            jax.ShapeDtypeStruct((N, D), jnp.float32),
            jax.ShapeDtypeStruct((N, 1), jnp.float32),
            jax.ShapeDtypeStruct((N, 1), jnp.float32),
        ],
    )(adj, h, W_down, b_down.reshape(1, D), W_pool, b_pool.reshape(1, 1))


_RBM = 200
_NP = 10240


def _rank_body(scol_ref, srow_ref, rank_ref):
    i0 = pl.program_id(0) * _RBM
    scol = scol_ref[...]
    srow = srow_ref[...]
    jrow = jax.lax.broadcasted_iota(jnp.int32, (_RBM, _NP), 1)
    icol = jax.lax.broadcasted_iota(jnp.int32, (_RBM, _NP), 0) + i0
    above = (srow > scol) | ((srow == scol) & (jrow < icol))
    rank = jnp.sum(jnp.where(above, 1.0, 0.0), axis=1, keepdims=True)
    rank_ref[...] = rank.astype(jnp.int32)


def _rank_of(scol, srow):
    return pl.pallas_call(
        _rank_body,
        grid=(N // _RBM,),
        in_specs=[
            pl.BlockSpec((_RBM, 1), lambda i: (i, 0)),
            pl.BlockSpec((1, _NP), lambda i: (0, 0)),
        ],
        out_specs=pl.BlockSpec((_RBM, 1), lambda i: (i, 0)),
        out_shape=jax.ShapeDtypeStruct((N, 1), jnp.int32),
    )(scol, srow)


_KP = 5120


def _topk_body(rank_hbm, sc_hbm, inv_hbm, h1_hbm,
               sidx_hbm, sval_hbm, sinv_hbm,
               rkb, scb, invb, i2d, n2d, v2d, w2d, idxg, rowg, sem):
    c = lax.axis_index("c")
    s = lax.axis_index("s")
    lanes = lax.iota(jnp.int32, 16)
    base = s * 640
    pltpu.sync_copy(rank_hbm.at[pl.ds(base, 640)], rkb)
    pltpu.sync_copy(sc_hbm.at[pl.ds(base, 640)], scb)
    pltpu.sync_copy(inv_hbm.at[pl.ds(base, 640)], invb)

    @pl.when(s == 0)
    def _padfill():
        def _pf(i, carry):
            i2d[0, pl.ds(i * 16, 16)] = jnp.zeros((16,), jnp.int32)
            return carry
        lax.fori_loop(0, 8, _pf, 0)
        pltpu.sync_copy(i2d.at[0, pl.ds(0, 120)], sidx_hbm.at[pl.ds(K, 120)])

    def _it(i, carry):
        sl = pl.ds(i * 16, 16)
        r = rkb[sl]
        sel = r < K
        rd = jnp.where(sel, r, K + 8 + lanes)
        row = i >> 3
        col = (i & 7) * 16
        i2d[row, pl.ds(col, 16)] = rd
        n2d[row, pl.ds(col, 16)] = base + i * 16 + lanes
        v2d[row, pl.ds(col, 16)] = scb[sl]
        w2d[row, pl.ds(col, 16)] = invb[sl]
        return carry
    lax.fori_loop(0, 40, _it, 0)

    def _fl(b, carry):
        pltpu.sync_copy(n2d.at[b], sidx_hbm.at[i2d.at[b]])
        pltpu.sync_copy(v2d.at[b], sval_hbm.at[i2d.at[b]])
        pltpu.sync_copy(w2d.at[b], sinv_hbm.at[i2d.at[b]])
        return carry
    lax.fori_loop(0, 5, _fl, 0)
    plsc.subcore_barrier()

    del h1_hbm, idxg, rowg, sem, c


@jax.jit
def _topk_arrays(rank_pad, sc_pad, inv_pad, h1):
    mesh = plsc.VectorSubcoreMesh(core_axis_name="c", subcore_axis_name="s")
    f = pl.kernel(
        _topk_body,
        out_type=(
            jax.ShapeDtypeStruct((_KP,), jnp.int32),
            jax.ShapeDtypeStruct((_KP,), jnp.float32),
            jax.ShapeDtypeStruct((_KP,), jnp.float32),
        ),
        mesh=mesh,
        scratch_types=[
            pltpu.VMEM((640,), jnp.int32),
            pltpu.VMEM((640,), jnp.float32),
            pltpu.VMEM((640,), jnp.float32),
            pltpu.VMEM((5, 128), jnp.int32),
            pltpu.VMEM((5, 128), jnp.int32),
            pltpu.VMEM((5, 128), jnp.float32),
            pltpu.VMEM((5, 128), jnp.float32),
            pltpu.VMEM((2, 80), jnp.int32),
            pltpu.VMEM((80, D), jnp.float32),
            pltpu.SemaphoreType.DMA,
        ],
        compiler_params=pltpu.CompilerParams(needs_layout_passes=False),
    )
    return f(rank_pad, sc_pad, inv_pad, h1)


def _rowgather_body(tab_hbm, idx_hbm, out_hbm, idxv, rowg, sem):
    c = lax.axis_index("c")
    s = lax.axis_index("s")
    wid = s * 2 + c
    for j in range(2):
        pltpu.sync_copy(idx_hbm.at[pl.ds(wid * 160 + j * 80, 80)], idxv.at[j])
        pltpu.async_copy(tab_hbm.at[idxv.at[j]], rowg, sem).wait()
        pltpu.sync_copy(rowg, out_hbm.at[pl.ds(wid * 160 + j * 80, 80)])


@jax.jit
def _rowgather(tab, idx):
    mesh = plsc.VectorSubcoreMesh(core_axis_name="c", subcore_axis_name="s")
    f = pl.kernel(
        _rowgather_body,
        out_type=jax.ShapeDtypeStruct((_KP, D), jnp.float32),
        mesh=mesh,
        scratch_types=[
            pltpu.VMEM((2, 80), jnp.int32),
            pltpu.VMEM((80, D), jnp.float32),
            pltpu.SemaphoreType.DMA,
        ],
        compiler_params=pltpu.CompilerParams(needs_layout_passes=False),
    )
    return f(tab, idx)


def _rowgw_body(tab_hbm, idx_hbm, out_hbm, idxv, rowg, sem):
    c = lax.axis_index("c")
    s = lax.axis_index("s")
    wid = s * 2 + c
    for j in range(2):
        pltpu.sync_copy(idx_hbm.at[pl.ds(wid * 160 + j * 80, 80)], idxv.at[j])
        for o in range(10):
            pltpu.async_copy(tab_hbm.at[idxv.at[j, pl.ds(o * 8, 8)]], rowg, sem).wait()
            pltpu.sync_copy(rowg, out_hbm.at[pl.ds(wid * 160 + j * 80 + o * 8, 8)])


@jax.jit
def _rowgather_wide(tab, idx):
    mesh = plsc.VectorSubcoreMesh(core_axis_name="c", subcore_axis_name="s")
    f = pl.kernel(
        _rowgw_body,
        out_type=jax.ShapeDtypeStruct((_KP, _NP), jnp.float32),
        mesh=mesh,
        scratch_types=[
            pltpu.VMEM((2, 80), jnp.int32),
            pltpu.VMEM((8, _NP), jnp.float32),
            pltpu.SemaphoreType.DMA,
        ],
        compiler_params=pltpu.CompilerParams(needs_layout_passes=False),
    )
    return f(tab, idx)


def _rowgn_body(tab_hbm, idx_hbm, out_hbm, idxv, rowg, sem):
    c = lax.axis_index("c")
    s = lax.axis_index("s")
    wid = s * 2 + c
    for j in range(4):
        pltpu.sync_copy(idx_hbm.at[pl.ds(wid * 320 + j * 80, 80)], idxv.at[j])
        pltpu.async_copy(tab_hbm.at[idxv.at[j]], rowg, sem).wait()
        pltpu.sync_copy(rowg, out_hbm.at[pl.ds(wid * 320 + j * 80, 80)])


@jax.jit
def _rowgather_n(tab, idx):
    mesh = plsc.VectorSubcoreMesh(core_axis_name="c", subcore_axis_name="s")
    f = pl.kernel(
        _rowgn_body,
        out_type=jax.ShapeDtypeStruct((_NP, D), jnp.float32),
        mesh=mesh,
        scratch_types=[
            pltpu.VMEM((4, 80), jnp.int32),
            pltpu.VMEM((80, D), jnp.float32),
            pltpu.SemaphoreType.DMA,
        ],
        compiler_params=pltpu.CompilerParams(needs_layout_passes=False),
    )
    return f(tab, idx)


def _h2_body(g_ref, v_ref, h2_ref):
    h2_ref[...] = g_ref[...] * v_ref[...]


def _h2_scale(gath, vals_col):
    return pl.pallas_call(
        _h2_body,
        grid=(25,),
        in_specs=[pl.BlockSpec((200, D), lambda i: (i, 0)),
                  pl.BlockSpec((200, 1), lambda i: (i, 0))],
        out_specs=pl.BlockSpec((200, D), lambda i: (i, 0)),
        out_shape=jax.ShapeDtypeStruct((K, D), jnp.float32),
    )(gath, vals_col)


def _att_body(h2b_ref, h2_ref, att_ref, av_ref):
    sl = jnp.dot(h2b_ref[...], h2_ref[...].T) / jnp.sqrt(jnp.float32(D))
    m = jnp.max(sl, axis=1, keepdims=True)
    e = jnp.exp(sl - m)
    att = e / jnp.sum(e, axis=1, keepdims=True)
    att_ref[...] = att
    av_ref[...] = jnp.dot(att, h2_ref[...])


def _att_av(h2):
    return pl.pallas_call(
        _att_body,
        grid=(25,),
        in_specs=[pl.BlockSpec((200, D), lambda i: (i, 0)),
                  pl.BlockSpec((K, D), lambda i: (0, 0))],
        out_specs=[pl.BlockSpec((200, K), lambda i: (i, 0)),
                   pl.BlockSpec((200, D), lambda i: (i, 0))],
        out_shape=[jax.ShapeDtypeStruct((K, K), jnp.float32),
                   jax.ShapeDtypeStruct((K, D), jnp.float32)],
    )(h2, h2)


def _h3_body(ar_ref, p2_ref, rank_ref, av_ref, iv_ref, wb_ref, bb_ref, h3_ref):
    selcol = jnp.where(rank_ref[...] < K, 1.0, 0.0)
    ar = ar_ref[...]
    iv = iv_ref[...]
    craw = jnp.dot(ar, selcol) * iv
    s2 = jnp.dot(ar, p2_ref[...])
    num = s2 * iv / (craw + 1e-8) + av_ref[...]
    den = craw / (craw + 1e-8) + 1.0 + 1e-8
    x = num / den
    h3_ref[...] = jnp.maximum(jnp.dot(x, wb_ref[...]) + bb_ref[...], 0.0)


def _h3_of(arows, p2, rank_col, av, iv_col, W_bot, b_bot):
    return pl.pallas_call(
        _h3_body,
        grid=(25,),
        in_specs=[pl.BlockSpec((200, _NP), lambda i: (i, 0)),
                  pl.BlockSpec((_NP, D), lambda i: (0, 0)),
                  pl.BlockSpec((_NP, 1), lambda i: (0, 0)),
                  pl.BlockSpec((200, D), lambda i: (i, 0)),
                  pl.BlockSpec((200, 1), lambda i: (i, 0)),
                  pl.BlockSpec((D, D), lambda i: (0, 0)),
                  pl.BlockSpec((1, D), lambda i: (0, 0))],
        out_specs=pl.BlockSpec((200, D), lambda i: (i, 0)),
        out_shape=jax.ShapeDtypeStruct((K, D), jnp.float32),
    )(arows, p2, rank_col, av, iv_col, W_bot, b_bot.reshape(1, D))


def _fin_body(adj_ref, h1_ref, p3_ref, iv_ref, wu_ref, bu_ref, out_ref):
    hup = h1_ref[...] + p3_ref[...]
    s3 = jnp.dot(adj_ref[...], hup) * iv_ref[...]
    out_ref[...] = jnp.dot(s3, wu_ref[...]) + bu_ref[...]


def _out_of(adj, h1, p3, inv_col, W_up, b_up):
    nc = W_up.shape[1]
    return pl.pallas_call(
        _fin_body,
        grid=(50,),
        in_specs=[pl.BlockSpec((200, _NP), lambda i: (i, 0)),
                  pl.BlockSpec((_NP, D), lambda i: (0, 0)),
                  pl.BlockSpec((_NP, D), lambda i: (0, 0)),
                  pl.BlockSpec((200, 1), lambda i: (i, 0)),
                  pl.BlockSpec((D, nc), lambda i: (0, 0)),
                  pl.BlockSpec((1, nc), lambda i: (0, 0))],
        out_specs=pl.BlockSpec((200, nc), lambda i: (i, 0)),
        out_shape=jax.ShapeDtypeStruct((N, nc), jnp.float32),
    )(adj, h1, p3, inv_col, W_up, b_up.reshape(1, nc))


def kernel(h, edge_index, W_down0, b_down0, W_pool0, b_pool0, W_bot, b_bot, W_up0, b_up0):
    src, dst = edge_index[0], edge_index[1]
    adjp = _adj_counts(src, dst).reshape(N, _NP)
    g, h1, sc, inv = _dense_scores(adjp, h, W_down0, b_down0, W_pool0, b_pool0)
    scores = sc.reshape(N)
    srow = jnp.pad(scores, (0, _NP - N), constant_values=-1.0).reshape(1, _NP)
    rank = _rank_of(sc, srow)
    rank_pad = jnp.pad(rank.reshape(N), (0, _NP - N), constant_values=1 << 30)
    sc_pad = jnp.pad(scores, (0, _NP - N))
    inv_pad = jnp.pad(inv.reshape(N), (0, _NP - N))
    sidx, sval, sinv = _topk_arrays(rank_pad, sc_pad, inv_pad, h1)
    gath = _rowgather(h1, sidx)
    h2 = _h2_scale(gath[:K], sval[:K].reshape(K, 1))
    att, av = _att_av(h2)
    arows = _rowgather_wide(adjp, sidx)
    rankc = jnp.minimum(rank_pad, K)
    h2z = jnp.concatenate([h2, jnp.zeros((8, D), jnp.float32)], axis=0)
    p2 = _rowgather_n(h2z, rankc)
    h3 = _h3_of(arows[:K], p2, rank_pad.reshape(_NP, 1), av,
                sinv[:K].reshape(K, 1), W_bot, b_bot)
    h3z = jnp.concatenate([h3, jnp.zeros((8, D), jnp.float32)], axis=0)
    p3 = _rowgather_n(h3z, rankc)
    h1p = jnp.pad(h1, ((0, _NP - N), (0, 0)))
    out = _out_of(adjp, h1p, p3, inv, W_up0, b_up0)
    return (out, att, h2, g)


# speedup vs baseline: 1.0378x; 1.0378x over previous
import functools

import jax
import jax.numpy as jnp
from jax import lax
from jax.experimental import pallas as pl
from jax.experimental.pallas import tpu as pltpu
from jax.experimental.pallas import tpu_sc as plsc

N = 10000
E = 160000
D = 128
K = 5000


_WSC = 1_280_000
_GWIN = _WSC + 128
_NPASS = N * 10240 // (2 * _WSC)
_ECH = E // 16
_WCH = 20_000
_NCH = _WSC // _WCH


def _adj_body(src_hbm, dst_hbm, g_hbm, gwin, keyb, dstc, stage2d,
              ones_v, zeros_v, wbuf):
    c = lax.axis_index("c")
    s = lax.axis_index("s")
    lanes = lax.iota(jnp.int32, 16)

    def _zb(i, carry):
        wbuf[pl.ds(i * 16, 16)] = jnp.zeros((16,), jnp.float32)
        return carry
    lax.fori_loop(0, _WCH // 16, _zb, 0)
    for j in range(8):
        ones_v[pl.ds(j * 16, 16)] = jnp.ones((16,), jnp.float32)
        zeros_v[pl.ds(j * 16, 16)] = jnp.zeros((16,), jnp.float32)

    pltpu.sync_copy(src_hbm.at[pl.ds(s * _ECH, _ECH)], keyb)
    for j in range(5):
        pltpu.sync_copy(dst_hbm.at[pl.ds(s * _ECH + j * 2000, 2000)], dstc)

        def _key(i, carry):
            keyb[pl.ds(j * 2000 + i * 16, 16)] = (
                keyb[pl.ds(j * 2000 + i * 16, 16)] * _NP + dstc[pl.ds(i * 16, 16)])
            return carry
        lax.fori_loop(0, 2000 // 16, _key, 0)

    for j in range(4):
        @pl.when(s + 16 * j < _NCH)
        def _z():
            pltpu.sync_copy(wbuf, gwin.at[pl.ds((s + 16 * j) * _WCH, _WCH)])
    @pl.when(s == 0)
    def _zd():
        pltpu.sync_copy(wbuf.at[pl.ds(0, 128)], gwin.at[pl.ds(_WSC, 128)])
    plsc.subcore_barrier()

    def _pass(p, carry):
        base = p * (2 * _WSC) + c * _WSC

        def _scan(i, cnt):
            k = keyb[pl.ds(i * 16, 16)]
            t = k - base
            m = (t >= 0) & (t < _WSC)
            n_vec = plsc.all_reduce_population_count(m)
            _, tc = plsc.sort_key_val(jnp.where(m, 0, 1), t)
            pos = cnt + lanes
            plsc.store_scatter(stage2d, [pos >> 7, pos & 127], tc)
            return cnt + n_vec[0]
        cnt = lax.fori_loop(0, _ECH // 16, _scan, 0)

        for j in range(8):
            pos = cnt + j * 16 + lanes
            plsc.store_scatter(stage2d, [pos >> 7, pos & 127],
                               _WSC + j * 16 + lanes)
        nb = (cnt + 127) // 128

        def _sc(b, carry):
            pltpu.sync_copy(ones_v, gwin.at[stage2d.at[b]], add=True)
            return carry
        lax.fori_loop(0, nb, _sc, 0)
        plsc.subcore_barrier()

        for j in range(4):
            @pl.when(s + 16 * j < _NCH)
            def _wb():
                off = (s + 16 * j) * _WCH
                pltpu.sync_copy(gwin.at[pl.ds(off, _WCH)], wbuf)
                pltpu.sync_copy(wbuf, g_hbm.at[pl.ds(base + off, _WCH)])
        plsc.subcore_barrier()

        def _rz(b, carry):
            pltpu.sync_copy(zeros_v, gwin.at[stage2d.at[b]])
            return carry
        lax.fori_loop(0, nb, _rz, 0)
        plsc.subcore_barrier()
        return carry

    lax.fori_loop(0, _NPASS, _pass, 0)


@jax.jit
def _adj_counts(src, dst):
    mesh = plsc.VectorSubcoreMesh(core_axis_name="c", subcore_axis_name="s")
    f = pl.kernel(
        _adj_body,
        out_type=jax.ShapeDtypeStruct((N * 10240,), jnp.float32),
        mesh=mesh,
        scratch_types=[
            pltpu.VMEM_SHARED((_GWIN,), jnp.float32),
            pltpu.VMEM((_ECH,), jnp.int32),
            pltpu.VMEM((2000,), jnp.int32),
            pltpu.VMEM((80, 128), jnp.int32),
            pltpu.VMEM((128,), jnp.float32),
            pltpu.VMEM((128,), jnp.float32),
            pltpu.VMEM((_WCH,), jnp.float32),
        ],
        compiler_params=pltpu.CompilerParams(needs_layout_passes=False),
    )
    return f(src, dst)


_BM = 200


def _score_body(adj_ref, h_ref, wd_ref, bd_ref, wp_ref, bp_ref,
                g_ref, h1_ref, sc_ref, inv_ref):
    adjp = adj_ref[...]
    adj = adjp[:, :N]
    deg = jnp.sum(adjp, axis=1, keepdims=True)
    g = adj / (deg + 1e-8)
    g_ref[...] = g
    inv_ref[...] = 1.0 / (deg + 1e-8)
    t = jnp.dot(g, h_ref[...])
    h1 = jnp.maximum(jnp.dot(t, wd_ref[...]) + bd_ref[...], 0.0)
    h1_ref[...] = h1
    sgt = jnp.dot(h1, wp_ref[...]) + bp_ref[...]
    sc_ref[...] = 1.0 / (1.0 + jnp.exp(-sgt))


def _dense_scores(adj, h, W_down, b_down, W_pool, b_pool):
    grid = N // _BM
    return pl.pallas_call(
        _score_body,
        grid=(grid,),
        in_specs=[
            pl.BlockSpec((_BM, 10240), lambda i: (i, 0)),
            pl.BlockSpec((N, D), lambda i: (0, 0)),
            pl.BlockSpec((D, D), lambda i: (0, 0)),
            pl.BlockSpec((1, D), lambda i: (0, 0)),
            pl.BlockSpec((D, 1), lambda i: (0, 0)),
            pl.BlockSpec((1, 1), lambda i: (0, 0)),
        ],
        out_specs=[
            pl.BlockSpec((_BM, N), lambda i: (i, 0)),
            pl.BlockSpec((_BM, D), lambda i: (i, 0)),
            pl.BlockSpec((_BM, 1), lambda i: (i, 0)),
            pl.BlockSpec((_BM, 1), lambda i: (i, 0)),
        ],
        out_shape=[
            jax.ShapeDtypeStruct((N, N), jnp.float32),
            jax.ShapeDtypeStruct((N, D), jnp.float32),
            jax.ShapeDtypeStruct((N, 1), jnp.float32),
            jax.ShapeDtypeStruct((N, 1), jnp.float32),
        ],
    )(adj, h, W_down, b_down.reshape(1, D), W_pool, b_pool.reshape(1, 1))


_RBM = 200
_NP = 10240


def _rank_body(scol_ref, srow_ref, rank_ref):
    i0 = pl.program_id(0) * _RBM
    scol = scol_ref[...]
    srow = srow_ref[...]
    jrow = jax.lax.broadcasted_iota(jnp.int32, (_RBM, _NP), 1)
    icol = jax.lax.broadcasted_iota(jnp.int32, (_RBM, _NP), 0) + i0
    above = (srow > scol) | ((srow == scol) & (jrow < icol))
    rank = jnp.sum(jnp.where(above, 1.0, 0.0), axis=1, keepdims=True)
    rank_ref[...] = rank.astype(jnp.int32)


def _rank_of(scol, srow):
    return pl.pallas_call(
        _rank_body,
        grid=(N // _RBM,),
        in_specs=[
            pl.BlockSpec((_RBM, 1), lambda i: (i, 0)),
            pl.BlockSpec((1, _NP), lambda i: (0, 0)),
        ],
        out_specs=pl.BlockSpec((_RBM, 1), lambda i: (i, 0)),
        out_shape=jax.ShapeDtypeStruct((N, 1), jnp.int32),
    )(scol, srow)


_KP = 5120


def _topk_body(rank_hbm, sc_hbm, inv_hbm, h1_hbm,
               sidx_hbm, sval_hbm, sinv_hbm,
               rkb, scb, invb, i2d, n2d, v2d, w2d, idxg, rowg, sem):
    c = lax.axis_index("c")
    s = lax.axis_index("s")
    lanes = lax.iota(jnp.int32, 16)
    base = s * 640
    pltpu.sync_copy(rank_hbm.at[pl.ds(base, 640)], rkb)
    pltpu.sync_copy(sc_hbm.at[pl.ds(base, 640)], scb)
    pltpu.sync_copy(inv_hbm.at[pl.ds(base, 640)], invb)

    @pl.when(s == 0)
    def _padfill():
        def _pf(i, carry):
            i2d[0, pl.ds(i * 16, 16)] = jnp.zeros((16,), jnp.int32)
            return carry
        lax.fori_loop(0, 8, _pf, 0)
        pltpu.sync_copy(i2d.at[0, pl.ds(0, 120)], sidx_hbm.at[pl.ds(K, 120)])

    def _it(i, carry):
        sl = pl.ds(i * 16, 16)
        r = rkb[sl]
        sel = r < K
        rd = jnp.where(sel, r, K + 8 + lanes)
        row = i >> 3
        col = (i & 7) * 16
        i2d[row, pl.ds(col, 16)] = rd
        n2d[row, pl.ds(col, 16)] = base + i * 16 + lanes
        v2d[row, pl.ds(col, 16)] = scb[sl]
        w2d[row, pl.ds(col, 16)] = invb[sl]
        return carry
    lax.fori_loop(0, 40, _it, 0)

    def _fl(b, carry):
        pltpu.sync_copy(n2d.at[b], sidx_hbm.at[i2d.at[b]])
        pltpu.sync_copy(v2d.at[b], sval_hbm.at[i2d.at[b]])
        pltpu.sync_copy(w2d.at[b], sinv_hbm.at[i2d.at[b]])
        return carry
    lax.fori_loop(0, 5, _fl, 0)
    plsc.subcore_barrier()

    del h1_hbm, idxg, rowg, sem, c


@jax.jit
def _topk_arrays(rank_pad, sc_pad, inv_pad, h1):
    mesh = plsc.VectorSubcoreMesh(core_axis_name="c", subcore_axis_name="s")
    f = pl.kernel(
        _topk_body,
        out_type=(
            jax.ShapeDtypeStruct((_KP,), jnp.int32),
            jax.ShapeDtypeStruct((_KP,), jnp.float32),
            jax.ShapeDtypeStruct((_KP,), jnp.float32),
        ),
        mesh=mesh,
        scratch_types=[
            pltpu.VMEM((640,), jnp.int32),
            pltpu.VMEM((640,), jnp.float32),
            pltpu.VMEM((640,), jnp.float32),
            pltpu.VMEM((5, 128), jnp.int32),
            pltpu.VMEM((5, 128), jnp.int32),
            pltpu.VMEM((5, 128), jnp.float32),
            pltpu.VMEM((5, 128), jnp.float32),
            pltpu.VMEM((2, 80), jnp.int32),
            pltpu.VMEM((80, D), jnp.float32),
            pltpu.SemaphoreType.DMA,
        ],
        compiler_params=pltpu.CompilerParams(needs_layout_passes=False),
    )
    return f(rank_pad, sc_pad, inv_pad, h1)


def _rowgather_body(tab_hbm, idx_hbm, out_hbm, idxv, rowg, sem):
    c = lax.axis_index("c")
    s = lax.axis_index("s")
    wid = s * 2 + c
    for j in range(2):
        pltpu.sync_copy(idx_hbm.at[pl.ds(wid * 160 + j * 80, 80)], idxv.at[j])
        pltpu.async_copy(tab_hbm.at[idxv.at[j]], rowg, sem).wait()
        pltpu.sync_copy(rowg, out_hbm.at[pl.ds(wid * 160 + j * 80, 80)])


@jax.jit
def _rowgather(tab, idx):
    mesh = plsc.VectorSubcoreMesh(core_axis_name="c", subcore_axis_name="s")
    f = pl.kernel(
        _rowgather_body,
        out_type=jax.ShapeDtypeStruct((_KP, D), jnp.float32),
        mesh=mesh,
        scratch_types=[
            pltpu.VMEM((2, 80), jnp.int32),
            pltpu.VMEM((80, D), jnp.float32),
            pltpu.SemaphoreType.DMA,
        ],
        compiler_params=pltpu.CompilerParams(needs_layout_passes=False),
    )
    return f(tab, idx)


def _rowgw_body(tab_hbm, idx_hbm, out_hbm, idxv, rowg, sem):
    c = lax.axis_index("c")
    s = lax.axis_index("s")
    wid = s * 2 + c
    for j in range(2):
        pltpu.sync_copy(idx_hbm.at[pl.ds(wid * 160 + j * 80, 80)], idxv.at[j])
        for o in range(10):
            pltpu.async_copy(tab_hbm.at[idxv.at[j, pl.ds(o * 8, 8)]], rowg, sem).wait()
            pltpu.sync_copy(rowg, out_hbm.at[pl.ds(wid * 160 + j * 80 + o * 8, 8)])


@jax.jit
def _rowgather_wide(tab, idx):
    mesh = plsc.VectorSubcoreMesh(core_axis_name="c", subcore_axis_name="s")
    f = pl.kernel(
        _rowgw_body,
        out_type=jax.ShapeDtypeStruct((_KP, _NP), jnp.float32),
        mesh=mesh,
        scratch_types=[
            pltpu.VMEM((2, 80), jnp.int32),
            pltpu.VMEM((8, _NP), jnp.float32),
            pltpu.SemaphoreType.DMA,
        ],
        compiler_params=pltpu.CompilerParams(needs_layout_passes=False),
    )
    return f(tab, idx)


def _rowgn_body(tab_hbm, idx_hbm, out_hbm, idxv, rowg, sem):
    c = lax.axis_index("c")
    s = lax.axis_index("s")
    wid = s * 2 + c
    for j in range(4):
        pltpu.sync_copy(idx_hbm.at[pl.ds(wid * 320 + j * 80, 80)], idxv.at[j])
        pltpu.async_copy(tab_hbm.at[idxv.at[j]], rowg, sem).wait()
        pltpu.sync_copy(rowg, out_hbm.at[pl.ds(wid * 320 + j * 80, 80)])


@jax.jit
def _rowgather_n(tab, idx):
    mesh = plsc.VectorSubcoreMesh(core_axis_name="c", subcore_axis_name="s")
    f = pl.kernel(
        _rowgn_body,
        out_type=jax.ShapeDtypeStruct((_NP, D), jnp.float32),
        mesh=mesh,
        scratch_types=[
            pltpu.VMEM((4, 80), jnp.int32),
            pltpu.VMEM((80, D), jnp.float32),
            pltpu.SemaphoreType.DMA,
        ],
        compiler_params=pltpu.CompilerParams(needs_layout_passes=False),
    )
    return f(tab, idx)


def _h2_body(g_ref, v_ref, h2_ref):
    h2_ref[...] = g_ref[...] * v_ref[...]


def _h2_scale(gath, vals_col):
    return pl.pallas_call(
        _h2_body,
        grid=(25,),
        in_specs=[pl.BlockSpec((200, D), lambda i: (i, 0)),
                  pl.BlockSpec((200, 1), lambda i: (i, 0))],
        out_specs=pl.BlockSpec((200, D), lambda i: (i, 0)),
        out_shape=jax.ShapeDtypeStruct((K, D), jnp.float32),
    )(gath, vals_col)


def _att_body(h2b_ref, h2_ref, att_ref, av_ref):
    sl = jnp.dot(h2b_ref[...], h2_ref[...].T) / jnp.sqrt(jnp.float32(D))
    m = jnp.max(sl, axis=1, keepdims=True)
    e = jnp.exp(sl - m)
    att = e / jnp.sum(e, axis=1, keepdims=True)
    att_ref[...] = att
    av_ref[...] = jnp.dot(att, h2_ref[...])


def _att_av(h2):
    return pl.pallas_call(
        _att_body,
        grid=(25,),
        in_specs=[pl.BlockSpec((200, D), lambda i: (i, 0)),
                  pl.BlockSpec((K, D), lambda i: (0, 0))],
        out_specs=[pl.BlockSpec((200, K), lambda i: (i, 0)),
                   pl.BlockSpec((200, D), lambda i: (i, 0))],
        out_shape=[jax.ShapeDtypeStruct((K, K), jnp.float32),
                   jax.ShapeDtypeStruct((K, D), jnp.float32)],
    )(h2, h2)


def _s2f_body(adj_ref, p2_ref, sel_ref, s2_ref, cr_ref):
    ad = adj_ref[...]
    s2_ref[...] = jnp.dot(ad, p2_ref[...])
    cr_ref[...] = jnp.dot(ad, sel_ref[...])


def _s2full(adjp, p2, sel_col):
    return pl.pallas_call(
        _s2f_body,
        grid=(50,),
        in_specs=[pl.BlockSpec((200, _NP), lambda i: (i, 0)),
                  pl.BlockSpec((_NP, D), lambda i: (0, 0)),
                  pl.BlockSpec((_NP, D), lambda i: (0, 0))],
        out_specs=[pl.BlockSpec((200, D), lambda i: (i, 0)),
                   pl.BlockSpec((200, D), lambda i: (i, 0))],
        out_shape=[jax.ShapeDtypeStruct((N, D), jnp.float32),
                   jax.ShapeDtypeStruct((N, D), jnp.float32)],
    )(adjp, p2, sel_col)


def _h3_body(s2_ref, av_ref, c_ref, iv_ref, wb_ref, bb_ref, h3_ref):
    cc = c_ref[...]
    num = s2_ref[...] * iv_ref[...] / (cc + 1e-8) + av_ref[...]
    den = cc / (cc + 1e-8) + 1.0 + 1e-8
    x = num / den
    h3_ref[...] = jnp.maximum(jnp.dot(x, wb_ref[...]) + bb_ref[...], 0.0)


def _h3_of(s2, av, c_col, iv_col, W_bot, b_bot):
    return pl.pallas_call(
        _h3_body,
        grid=(25,),
        in_specs=[pl.BlockSpec((200, D), lambda i: (i, 0)),
                  pl.BlockSpec((200, D), lambda i: (i, 0)),
                  pl.BlockSpec((200, 1), lambda i: (i, 0)),
                  pl.BlockSpec((200, 1), lambda i: (i, 0)),
                  pl.BlockSpec((D, D), lambda i: (0, 0)),
                  pl.BlockSpec((1, D), lambda i: (0, 0))],
        out_specs=pl.BlockSpec((200, D), lambda i: (i, 0)),
        out_shape=jax.ShapeDtypeStruct((K, D), jnp.float32),
    )(s2, av, c_col, iv_col, W_bot, b_bot.reshape(1, D))


def _fin_body(adj_ref, h1_ref, p3_ref, iv_ref, wu_ref, bu_ref, out_ref):
    hup = h1_ref[...] + p3_ref[...]
    s3 = jnp.dot(adj_ref[...], hup) * iv_ref[...]
    out_ref[...] = jnp.dot(s3, wu_ref[...]) + bu_ref[...]


def _out_of(adj, h1, p3, inv_col, W_up, b_up):
    nc = W_up.shape[1]
    return pl.pallas_call(
        _fin_body,
        grid=(50,),
        in_specs=[pl.BlockSpec((200, _NP), lambda i: (i, 0)),
                  pl.BlockSpec((_NP, D), lambda i: (0, 0)),
                  pl.BlockSpec((_NP, D), lambda i: (0, 0)),
                  pl.BlockSpec((200, 1), lambda i: (i, 0)),
                  pl.BlockSpec((D, nc), lambda i: (0, 0)),
                  pl.BlockSpec((1, nc), lambda i: (0, 0))],
        out_specs=pl.BlockSpec((200, nc), lambda i: (i, 0)),
        out_shape=jax.ShapeDtypeStruct((N, nc), jnp.float32),
    )(adj, h1, p3, inv_col, W_up, b_up.reshape(1, nc))


def kernel(h, edge_index, W_down0, b_down0, W_pool0, b_pool0, W_bot, b_bot, W_up0, b_up0):
    src, dst = edge_index[0], edge_index[1]
    adjp = _adj_counts(src, dst).reshape(N, _NP)
    g, h1, sc, inv = _dense_scores(adjp, h, W_down0, b_down0, W_pool0, b_pool0)
    scores = sc.reshape(N)
    srow = jnp.pad(scores, (0, _NP - N), constant_values=-1.0).reshape(1, _NP)
    rank = _rank_of(sc, srow)
    rank_pad = jnp.pad(rank.reshape(N), (0, _NP - N), constant_values=1 << 30)
    sc_pad = jnp.pad(scores, (0, _NP - N))
    inv_pad = jnp.pad(inv.reshape(N), (0, _NP - N))
    sidx, sval, sinv = _topk_arrays(rank_pad, sc_pad, inv_pad, h1)
    gath = _rowgather(h1, sidx)
    h2 = _h2_scale(gath[:K], sval[:K].reshape(K, 1))
    att, av = _att_av(h2)
    rankc = jnp.minimum(rank_pad, K)
    h2z = jnp.concatenate([h2, jnp.zeros((8, D), jnp.float32)], axis=0)
    p2 = _rowgather_n(h2z, rankc)
    sel_col = jnp.broadcast_to(
        jnp.where(rank_pad < K, 1.0, 0.0).reshape(_NP, 1), (_NP, D))
    sel_col = jnp.asarray(sel_col)
    s2f, crep = _s2full(adjp, p2, sel_col)
    s2sel = _rowgather(s2f, sidx)
    crepsel = _rowgather(crep, sidx)
    craw = crepsel[:K, :1] * sinv[:K].reshape(K, 1)
    h3 = _h3_of(s2sel[:K], av, craw, sinv[:K].reshape(K, 1), W_bot, b_bot)
    h3z = jnp.concatenate([h3, jnp.zeros((8, D), jnp.float32)], axis=0)
    p3 = _rowgather_n(h3z, rankc)
    h1p = jnp.pad(h1, ((0, _NP - N), (0, 0)))
    out = _out_of(adjp, h1p, p3, inv, W_up0, b_up0)
    return (out, att, h2, g)
